# baseline (speedup 1.0000x reference)
import jax
import jax.numpy as jnp
from jax import lax
from jax.experimental import pallas as pl
from jax.experimental.pallas import tpu as pltpu
from jax.experimental.pallas import tpu_sc as plsc

VOCAB = 1000000
HIDDEN = 64
MAX_POS = 512
BATCH = 4096
SEQ = 200
EPS = 1e-12

NC = 2
NS = 16
NW = NC * NS
NTOK = BATCH * SEQ
TPW = NTOK // NW
CHUNK = 640
NCHUNK = TPW // CHUNK
IDX_W = 128
NSUB = CHUNK // IDX_W


def _rsqrt_newton(v):
    i = lax.bitcast_convert_type(v, jnp.int32)
    i = jnp.int32(0x5F3759DF) - lax.shift_right_arithmetic(i, 1)
    y = lax.bitcast_convert_type(i, jnp.float32)
    half = v * 0.5
    for _ in range(3):
        y = y * (1.5 - half * y * y)
    return y


def _sc_body(wid_hbm, pid_hbm, wtab_hbm, ptab_hbm, gam_hbm, bet_hbm, out_hbm,
             widx, pidx, wrows, prows, gbuf, bbuf, sem, semi):
    w = lax.axis_index("s") * NC + lax.axis_index("c")
    base_row = w * (TPW // IDX_W)

    pltpu.sync_copy(gam_hbm, gbuf)
    pltpu.sync_copy(bet_hbm, bbuf)
    gvs = [gbuf[pl.ds(ci * 16, 16)] for ci in range(HIDDEN // 16)]
    bvs = [bbuf[pl.ds(ci * 16, 16)] for ci in range(HIDDEN // 16)]

    lane = lax.iota(jnp.int32, 16)
    perms = [lax.bitwise_xor(lane, jnp.int32(1 << k)) for k in range(4)]

    def chunk_body(c, carry):
        r0 = base_row + c * NSUB
        t0 = w * TPW + c * CHUNK
        cpi1 = pltpu.async_copy(wid_hbm.at[pl.ds(r0, NSUB)], widx, semi)
        cpi2 = pltpu.async_copy(pid_hbm.at[pl.ds(r0, NSUB)], pidx, semi)
        cpi1.wait()
        cpi2.wait()
        cps = []
        for j in range(NSUB):
            cps.append(pltpu.async_copy(
                wtab_hbm.at[widx.at[j]],
                wrows.at[pl.ds(j * IDX_W, IDX_W)], sem))
            cps.append(pltpu.async_copy(
                ptab_hbm.at[pidx.at[j]],
                prows.at[pl.ds(j * IDX_W, IDX_W)], sem))
        for cp in cps:
            cp.wait()

        def tok_body(t, carry2):
            xs = []
            for ci in range(HIDDEN // 16):
                xs.append(wrows[t, pl.ds(ci * 16, 16)] +
                          prows[t, pl.ds(ci * 16, 16)])
            acc = (xs[0] + xs[1]) + (xs[2] + xs[3])
            sq = xs[0] * xs[0]
            for ci in range(1, HIDDEN // 16):
                sq = sq + xs[ci] * xs[ci]
            for pm in perms:
                acc = acc + acc.at[pm].get(mode="promise_in_bounds")
                sq = sq + sq.at[pm].get(mode="promise_in_bounds")
            mean = acc * (1.0 / HIDDEN)
            var = sq * (1.0 / HIDDEN) - mean * mean
            inv = _rsqrt_newton(var + EPS)
            for ci in range(HIDDEN // 16):
                wrows[t, pl.ds(ci * 16, 16)] = (
                    (xs[ci] - mean) * inv * gvs[ci] + bvs[ci])
            return carry2

        lax.fori_loop(0, CHUNK, tok_body, 0, unroll=2)
        pltpu.sync_copy(wrows, out_hbm.at[pl.ds(t0, CHUNK)])
        return carry

    lax.fori_loop(0, NCHUNK, chunk_body, 0)


@jax.jit
def _run(word_ids2d, posi_ids2d, word_table, posi_table, ln_gamma, ln_beta):
    mesh = plsc.VectorSubcoreMesh(core_axis_name="c", subcore_axis_name="s")
    f = pl.kernel(
        _sc_body,
        out_type=jax.ShapeDtypeStruct((NTOK, HIDDEN), jnp.float32),
        mesh=mesh,
        compiler_params=pltpu.CompilerParams(use_tc_tiling_on_sc=False),
        scratch_types=[
            pltpu.VMEM((NSUB, IDX_W), jnp.int32),
            pltpu.VMEM((NSUB, IDX_W), jnp.int32),
            pltpu.VMEM((CHUNK, HIDDEN), jnp.float32),
            pltpu.VMEM((CHUNK, HIDDEN), jnp.float32),
            pltpu.VMEM((HIDDEN,), jnp.float32),
            pltpu.VMEM((HIDDEN,), jnp.float32),
            pltpu.SemaphoreType.DMA,
            pltpu.SemaphoreType.DMA,
        ],
    )
    return f(word_ids2d, posi_ids2d, word_table, posi_table, ln_gamma, ln_beta)


def kernel(word_ids, posi_ids, word_table, posi_table, ln_gamma, ln_beta):
    wid2 = word_ids.reshape(NTOK // IDX_W, IDX_W).astype(jnp.int32)
    pid2 = posi_ids.reshape(NTOK // IDX_W, IDX_W).astype(jnp.int32)
    out = _run(wid2, pid2, word_table, posi_table, ln_gamma, ln_beta)
    return out.reshape(BATCH, SEQ, HIDDEN)


# speedup vs baseline: 1.0179x; 1.0179x over previous
import jax
import jax.numpy as jnp
from jax import lax
from jax.experimental import pallas as pl
from jax.experimental.pallas import tpu as pltpu
from jax.experimental.pallas import tpu_sc as plsc

VOCAB = 1000000
HIDDEN = 64
MAX_POS = 512
BATCH = 4096
SEQ = 200
EPS = 1e-12

NC = 2
NS = 16
NW = NC * NS
NTOK = BATCH * SEQ
TPW = NTOK // NW
CHUNK = 512
NCHUNK = TPW // CHUNK
IDX_W = 128
NSUB = CHUNK // IDX_W


def _rsqrt_newton(v):
    i = lax.bitcast_convert_type(v, jnp.int32)
    i = jnp.int32(0x5F3759DF) - lax.shift_right_arithmetic(i, 1)
    y = lax.bitcast_convert_type(i, jnp.float32)
    half = v * 0.5
    for _ in range(2):
        y = y * (1.5 - half * y * y)
    return y


def _sc_body(wid_hbm, pid_hbm, wtab_hbm, ptab_hbm, gam_hbm, bet_hbm, out_hbm,
             widx, pidx, wrows, prows, obuf, gbuf, bbuf, sem, semi):
    w = lax.axis_index("s") * NC + lax.axis_index("c")
    base_row = w * (TPW // IDX_W)

    pltpu.sync_copy(gam_hbm, gbuf)
    pltpu.sync_copy(bet_hbm, bbuf)
    gvs = [gbuf[pl.ds(ci * 16, 16)] for ci in range(HIDDEN // 16)]
    bvs = [bbuf[pl.ds(ci * 16, 16)] for ci in range(HIDDEN // 16)]

    lane = lax.iota(jnp.int32, 16)
    perms = [lax.bitwise_xor(lane, jnp.int32(1 << k)) for k in range(4)]

    def chunk_body(c, carry):
        r0 = base_row + c * NSUB
        t0 = w * TPW + c * CHUNK
        cpi1 = pltpu.async_copy(wid_hbm.at[pl.ds(r0, NSUB)], widx, semi)
        cpi2 = pltpu.async_copy(pid_hbm.at[pl.ds(r0, NSUB)], pidx, semi)
        cpi1.wait()
        cpi2.wait()
        cps = []
        for j in range(NSUB):
            cps.append(pltpu.async_copy(
                wtab_hbm.at[widx.at[j]],
                wrows.at[pl.ds(j * IDX_W, IDX_W)], sem))
            cps.append(pltpu.async_copy(
                ptab_hbm.at[pidx.at[j]],
                prows.at[pl.ds(j * IDX_W, IDX_W)], sem))
        for cp in cps:
            cp.wait()

        def tok_body(t, carry2):
            xs = []
            for ci in range(HIDDEN // 16):
                xs.append(wrows[t, pl.ds(ci * 16, 16)] +
                          prows[t, pl.ds(ci * 16, 16)])
            acc = (xs[0] + xs[1]) + (xs[2] + xs[3])
            sq = xs[0] * xs[0]
            for ci in range(1, HIDDEN // 16):
                sq = sq + xs[ci] * xs[ci]
            for pm in perms:
                acc = acc + acc.at[pm].get(mode="promise_in_bounds")
                sq = sq + sq.at[pm].get(mode="promise_in_bounds")
            mean = acc * (1.0 / HIDDEN)
            var = sq * (1.0 / HIDDEN) - mean * mean
            inv = _rsqrt_newton(var + EPS)
            for ci in range(HIDDEN // 16):
                obuf[t, pl.ds(ci * 16, 16)] = (
                    (xs[ci] - mean) * inv * gvs[ci] + bvs[ci])
            return carry2

        lax.fori_loop(0, CHUNK, tok_body, 0, unroll=2)
        pltpu.sync_copy(obuf, out_hbm.at[pl.ds(t0, CHUNK)])
        return carry

    lax.fori_loop(0, NCHUNK, chunk_body, 0)


@jax.jit
def _run(word_ids2d, posi_ids2d, word_table, posi_table, ln_gamma, ln_beta):
    mesh = plsc.VectorSubcoreMesh(core_axis_name="c", subcore_axis_name="s")
    f = pl.kernel(
        _sc_body,
        out_type=jax.ShapeDtypeStruct((NTOK, HIDDEN), jnp.float32),
        mesh=mesh,
        compiler_params=pltpu.CompilerParams(use_tc_tiling_on_sc=False),
        scratch_types=[
            pltpu.VMEM((NSUB, IDX_W), jnp.int32),
            pltpu.VMEM((NSUB, IDX_W), jnp.int32),
            pltpu.VMEM((CHUNK, HIDDEN), jnp.float32),
            pltpu.VMEM((CHUNK, HIDDEN), jnp.float32),
            pltpu.VMEM((CHUNK, HIDDEN), jnp.float32),
            pltpu.VMEM((HIDDEN,), jnp.float32),
            pltpu.VMEM((HIDDEN,), jnp.float32),
            pltpu.SemaphoreType.DMA,
            pltpu.SemaphoreType.DMA,
        ],
    )
    return f(word_ids2d, posi_ids2d, word_table, posi_table, ln_gamma, ln_beta)


def kernel(word_ids, posi_ids, word_table, posi_table, ln_gamma, ln_beta):
    wid2 = word_ids.reshape(NTOK // IDX_W, IDX_W).astype(jnp.int32)
    pid2 = posi_ids.reshape(NTOK // IDX_W, IDX_W).astype(jnp.int32)
    out = _run(wid2, pid2, word_table, posi_table, ln_gamma, ln_beta)
    return out.reshape(BATCH, SEQ, HIDDEN)


# speedup vs baseline: 1.4446x; 1.4192x over previous
import jax
import jax.numpy as jnp
from jax import lax
from jax.experimental import pallas as pl
from jax.experimental.pallas import tpu as pltpu
from jax.experimental.pallas import tpu_sc as plsc

VOCAB = 1000000
HIDDEN = 64
MAX_POS = 512
BATCH = 4096
SEQ = 200
EPS = 1e-12

NC = 2
NS = 16
NW = NC * NS
NTOK = BATCH * SEQ
TPW = NTOK // NW
CHUNK = 512
NCHUNK = TPW // CHUNK
IDX_W = 128
NSUB = CHUNK // IDX_W


def _rsqrt_newton(v):
    i = lax.bitcast_convert_type(v, jnp.int32)
    i = jnp.int32(0x5F3759DF) - lax.shift_right_arithmetic(i, 1)
    y = lax.bitcast_convert_type(i, jnp.float32)
    half = v * 0.5
    for _ in range(3):
        y = y * (1.5 - half * y * y)
    return y


def _sc_body(wid_hbm, pid_hbm, wtab_hbm, ptab_hbm, gam_hbm, bet_hbm, out_hbm,
             widx, pidx, wrows, prows, obuf, gbuf, bbuf, sem, semi):
    w = lax.axis_index("s") * NC + lax.axis_index("c")
    base_row = w * (TPW // IDX_W)

    pltpu.sync_copy(gam_hbm, gbuf)
    pltpu.sync_copy(bet_hbm, bbuf)
    gvs = [gbuf[pl.ds(ci * 16, 16)] for ci in range(HIDDEN // 16)]
    bvs = [bbuf[pl.ds(ci * 16, 16)] for ci in range(HIDDEN // 16)]

    lane = lax.iota(jnp.int32, 16)
    perms = [lax.bitwise_xor(lane, jnp.int32(1 << k)) for k in range(4)]

    def chunk_body(c, carry):
        r0 = base_row + c * NSUB
        t0 = w * TPW + c * CHUNK
        cpi1 = pltpu.async_copy(wid_hbm.at[pl.ds(r0, NSUB)], widx, semi)
        cpi2 = pltpu.async_copy(pid_hbm.at[pl.ds(r0, NSUB)], pidx, semi)
        cpi1.wait()
        cpi2.wait()
        cps = []
        for j in range(NSUB):
            cps.append(pltpu.async_copy(
                wtab_hbm.at[widx.at[j]],
                wrows.at[pl.ds(j * IDX_W, IDX_W)], sem))
            cps.append(pltpu.async_copy(
                ptab_hbm.at[pidx.at[j]],
                prows.at[pl.ds(j * IDX_W, IDX_W)], sem))
        for cp in cps:
            cp.wait()

        def tok_body(t, carry2):
            xs = []
            for ci in range(HIDDEN // 16):
                xs.append(wrows[t, pl.ds(ci * 16, 16)] +
                          prows[t, pl.ds(ci * 16, 16)])
            acc = (xs[0] + xs[1]) + (xs[2] + xs[3])
            sq = xs[0] * xs[0]
            for ci in range(1, HIDDEN // 16):
                sq = sq + xs[ci] * xs[ci]
            for pm in perms:
                acc = acc + acc.at[pm].get(mode="promise_in_bounds")
                sq = sq + sq.at[pm].get(mode="promise_in_bounds")
            mean = acc * (1.0 / HIDDEN)
            var = sq * (1.0 / HIDDEN) - mean * mean
            inv = _rsqrt_newton(var + EPS)
            for ci in range(HIDDEN // 16):
                obuf[t, pl.ds(ci * 16, 16)] = (
                    (xs[ci] - mean) * inv * gvs[ci] + bvs[ci])
            return carry2

        lax.fori_loop(0, CHUNK, tok_body, 0)
        pltpu.sync_copy(obuf, out_hbm.at[pl.ds(t0, CHUNK)])
        return carry

    lax.fori_loop(0, NCHUNK, chunk_body, 0)


@jax.jit
def _run(word_ids2d, posi_ids2d, word_table, posi_table, ln_gamma, ln_beta):
    mesh = plsc.VectorSubcoreMesh(core_axis_name="c", subcore_axis_name="s")
    f = pl.kernel(
        _sc_body,
        out_type=jax.ShapeDtypeStruct((NTOK, HIDDEN), jnp.float32),
        mesh=mesh,
        compiler_params=pltpu.CompilerParams(use_tc_tiling_on_sc=False),
        scratch_types=[
            pltpu.VMEM((NSUB, IDX_W), jnp.int32),
            pltpu.VMEM((NSUB, IDX_W), jnp.int32),
            pltpu.VMEM((CHUNK, HIDDEN), jnp.float32),
            pltpu.VMEM((CHUNK, HIDDEN), jnp.float32),
            pltpu.VMEM((CHUNK, HIDDEN), jnp.float32),
            pltpu.VMEM((HIDDEN,), jnp.float32),
            pltpu.VMEM((HIDDEN,), jnp.float32),
            pltpu.SemaphoreType.DMA,
            pltpu.SemaphoreType.DMA,
        ],
    )
    return f(word_ids2d, posi_ids2d, word_table, posi_table, ln_gamma, ln_beta)


def kernel(word_ids, posi_ids, word_table, posi_table, ln_gamma, ln_beta):
    wid2 = word_ids.reshape(NTOK // IDX_W, IDX_W).astype(jnp.int32)
    pid2 = posi_ids.reshape(NTOK // IDX_W, IDX_W).astype(jnp.int32)
    out = _run(wid2, pid2, word_table, posi_table, ln_gamma, ln_beta)
    return out.reshape(BATCH, SEQ, HIDDEN)
